# baseline (speedup 1.0000x reference)
import functools

import jax
import jax.numpy as jnp
from jax import lax
from jax.experimental import pallas as pl
from jax.experimental.pallas import tpu as pltpu
from jax.experimental.pallas import tpu_sc as plsc

EPS = 1e-6
L = 16
CH = 128
NW = 32

_GATHER_DNUMS = lax.GatherDimensionNumbers(
    offset_dims=(), collapsed_slice_dims=(0,), start_index_map=(0,))


def _dyn_gather(x, idx):
    return lax.gather(
        x, idx[:, None], _GATHER_DNUMS, slice_sizes=(1,),
        mode=lax.GatherScatterMode.PROMISE_IN_BOUNDS)


def _rsqrt_newton(xv):
    i = lax.bitcast_convert_type(xv, jnp.int32)
    i = jnp.int32(0x5F3759DF) - lax.shift_right_arithmetic(i, jnp.int32(1))
    y = lax.bitcast_convert_type(i, jnp.float32)
    half = xv * 0.5
    for _ in range(2):
        y = y * (1.5 - half * y * y)
    return y


def _make_sc_kernel(n_rows, dim):
    rows_per_w = n_rows // NW
    chunks = rows_per_w // CH
    nk = dim // L
    mesh = plsc.VectorSubcoreMesh(core_axis_name="c", subcore_axis_name="s")

    @functools.partial(
        pl.kernel,
        out_type=jax.ShapeDtypeStruct((n_rows, dim), jnp.float32),
        mesh=mesh,
        scratch_types=[
            pltpu.VMEM((rows_per_w,), jnp.int32),
            pltpu.VMEM((CH, dim), jnp.float32),
            pltpu.VMEM((CH, dim), jnp.float32),
            pltpu.VMEM((CH, dim), jnp.float32),
            pltpu.VMEM((CH, dim), jnp.float32),
            pltpu.VMEM((dim,), jnp.float32),
            pltpu.VMEM((dim,), jnp.float32),
            pltpu.VMEM((CH,), jnp.float32),
            pltpu.VMEM((CH,), jnp.float32),
            pltpu.SemaphoreType.DMA,
            pltpu.SemaphoreType.DMA,
            pltpu.SemaphoreType.DMA,
            pltpu.SemaphoreType.DMA,
        ],
    )
    def kern(table_hbm, idx_hbm, gamma_hbm, beta_hbm, out_hbm,
             idx_v, gb0, gb1, ob0, ob1, gamma_v, beta_v,
             stats_m, stats_r, gs0, gs1, os0, os1):
        wid = lax.axis_index("s") * 2 + lax.axis_index("c")
        pltpu.sync_copy(idx_hbm.at[pl.ds(wid * rows_per_w, rows_per_w)],
                        idx_v)
        pltpu.sync_copy(gamma_hbm, gamma_v)
        pltpu.sync_copy(beta_hbm, beta_v)
        gv = [gamma_v[pl.ds(k * L, L)] for k in range(nk)]
        bv = [beta_v[pl.ds(k * L, L)] for k in range(nk)]
        inv_d = jnp.float32(1.0 / dim)
        row0 = wid * rows_per_w
        lane = lax.iota(jnp.int32, L)

        gbuf = (gb0, gb1)
        obuf = (ob0, ob1)
        gsem = (gs0, gs1)
        osem = (os0, os1)

        def issue_gather(c, b):
            pltpu.async_copy(table_hbm.at[idx_v.at[pl.ds(c * CH, CH)]],
                             gbuf[b], gsem[b])

        def wait_gather(b):
            pltpu.make_async_copy(table_hbm.at[pl.ds(0, CH)], gbuf[b],
                                  gsem[b]).wait()

        def issue_out(c, b):
            pltpu.async_copy(obuf[b],
                             out_hbm.at[pl.ds(row0 + c * CH, CH)], osem[b])

        def wait_out(b):
            pltpu.make_async_copy(obuf[b], out_hbm.at[pl.ds(row0, CH)],
                                  osem[b]).wait()

        perms = [lane ^ sh for sh in (8, 4, 2, 1)]

        def ln_row(gb, ob, r):
            xs = [gb[r, pl.ds(k * L, L)] for k in range(nk)]
            s01, s23 = xs[0] + xs[1], xs[2] + xs[3]
            s45, s67 = xs[4] + xs[5], xs[6] + xs[7]
            s = (s01 + s23) + (s45 + s67)
            sq = [x * x for x in xs]
            q01, q23 = sq[0] + sq[1], sq[2] + sq[3]
            q45, q67 = sq[4] + sq[5], sq[6] + sq[7]
            ss = (q01 + q23) + (q45 + q67)
            for p in perms:
                s = s + _dyn_gather(s, p)
                ss = ss + _dyn_gather(ss, p)
            mean = s * inv_d
            var = ss * inv_d - mean * mean
            rstd = _rsqrt_newton(var + EPS)
            for k in range(nk):
                ob[r, pl.ds(k * L, L)] = (
                    (xs[k] - mean) * rstd * gv[k] + bv[k])

        def compute(b):
            gb, ob = gbuf[b], obuf[b]

            def row_body(i, _):
                r = i * 4
                for u in range(4):
                    ln_row(gb, ob, r + u)
                return 0

            lax.fori_loop(0, CH // 4, row_body, 0)

        issue_gather(0, 0)
        issue_gather(1, 1)
        for c in (0, 1):
            b = c & 1
            wait_gather(b)
            compute(b)
            issue_gather(c + 2, b)
            issue_out(c, b)

        def pair_body(p, _):
            for b in (0, 1):
                c = 2 + p * 2 + b
                wait_gather(b)
                compute(b)
                issue_gather(c + 2, b)
                wait_out(b)
                issue_out(c, b)
            return 0

        lax.fori_loop(0, (chunks - 4) // 2, pair_body, 0)

        for c in (chunks - 2, chunks - 1):
            b = c & 1
            wait_gather(b)
            compute(b)
            wait_out(b)
            issue_out(c, b)
        wait_out(0)
        wait_out(1)

    return kern


def kernel(src, table, gamma, beta):
    b, s = src.shape
    v, d = table.shape
    idx = src.reshape(-1).astype(jnp.int32)
    n = idx.shape[0]
    out = _make_sc_kernel(n, d)(table, idx, gamma, beta)
    return out.reshape(b, s, d)


# speedup vs baseline: 1.0535x; 1.0535x over previous
import functools

import jax
import jax.numpy as jnp
from jax import lax
from jax.experimental import pallas as pl
from jax.experimental.pallas import tpu as pltpu
from jax.experimental.pallas import tpu_sc as plsc

EPS = 1e-6
L = 16
CH = 128
NW = 32

_GATHER_DNUMS = lax.GatherDimensionNumbers(
    offset_dims=(), collapsed_slice_dims=(0,), start_index_map=(0,))


def _dyn_gather(x, idx):
    return lax.gather(
        x, idx[:, None], _GATHER_DNUMS, slice_sizes=(1,),
        mode=lax.GatherScatterMode.PROMISE_IN_BOUNDS)


def _rsqrt_newton(xv):
    i = lax.bitcast_convert_type(xv, jnp.int32)
    i = jnp.int32(0x5F375A86) - lax.shift_right_arithmetic(i, jnp.int32(1))
    y = lax.bitcast_convert_type(i, jnp.float32)
    half = xv * 0.5
    for _ in range(2):
        y = y * (1.5 - half * y * y)
    return y


def _make_sc_kernel(n_rows, dim):
    rows_per_w = n_rows // NW
    chunks = rows_per_w // CH
    nk = dim // L
    mesh = plsc.VectorSubcoreMesh(core_axis_name="c", subcore_axis_name="s")

    @functools.partial(
        pl.kernel,
        out_type=jax.ShapeDtypeStruct((n_rows, dim), jnp.float32),
        mesh=mesh,
        scratch_types=[
            pltpu.VMEM((rows_per_w,), jnp.int32),
            pltpu.VMEM((CH, dim), jnp.float32),
            pltpu.VMEM((CH, dim), jnp.float32),
            pltpu.VMEM((CH, dim), jnp.float32),
            pltpu.VMEM((CH, dim), jnp.float32),
            pltpu.VMEM((dim,), jnp.float32),
            pltpu.VMEM((dim,), jnp.float32),
            pltpu.VMEM((CH,), jnp.float32),
            pltpu.VMEM((CH,), jnp.float32),
            pltpu.SemaphoreType.DMA,
            pltpu.SemaphoreType.DMA,
            pltpu.SemaphoreType.DMA,
            pltpu.SemaphoreType.DMA,
        ],
    )
    def kern(table_hbm, idx_hbm, gamma_hbm, beta_hbm, out_hbm,
             idx_v, gb0, gb1, ob0, ob1, gamma_v, beta_v,
             stats_m, stats_r, gs0, gs1, os0, os1):
        wid = lax.axis_index("s") * 2 + lax.axis_index("c")
        pltpu.sync_copy(idx_hbm.at[pl.ds(wid * rows_per_w, rows_per_w)],
                        idx_v)
        pltpu.sync_copy(gamma_hbm, gamma_v)
        pltpu.sync_copy(beta_hbm, beta_v)
        gv = [gamma_v[pl.ds(k * L, L)] for k in range(nk)]
        bv = [beta_v[pl.ds(k * L, L)] for k in range(nk)]
        inv_d = jnp.float32(1.0 / dim)
        row0 = wid * rows_per_w
        lane = lax.iota(jnp.int32, L)

        gbuf = (gb0, gb1)
        obuf = (ob0, ob1)
        gsem = (gs0, gs1)
        osem = (os0, os1)

        def issue_gather(c, b):
            pltpu.async_copy(table_hbm.at[idx_v.at[pl.ds(c * CH, CH)]],
                             gbuf[b], gsem[b])

        def wait_gather(b):
            pltpu.make_async_copy(table_hbm.at[pl.ds(0, CH)], gbuf[b],
                                  gsem[b]).wait()

        def issue_out(c, b):
            pltpu.async_copy(obuf[b],
                             out_hbm.at[pl.ds(row0 + c * CH, CH)], osem[b])

        def wait_out(b):
            pltpu.make_async_copy(obuf[b], out_hbm.at[pl.ds(row0, CH)],
                                  osem[b]).wait()

        perms = [lane ^ sh for sh in (8, 4, 2, 1)]

        def row_stats(gb, r, j, accm, accr):
            xs = [gb[r, pl.ds(k * L, L)] for k in range(nk)]
            s01, s23 = xs[0] + xs[1], xs[2] + xs[3]
            s45, s67 = xs[4] + xs[5], xs[6] + xs[7]
            s = (s01 + s23) + (s45 + s67)
            sq = [x * x for x in xs]
            q01, q23 = sq[0] + sq[1], sq[2] + sq[3]
            q45, q67 = sq[4] + sq[5], sq[6] + sq[7]
            ss = (q01 + q23) + (q45 + q67)
            for p in perms:
                s = s + _dyn_gather(s, p)
                ss = ss + _dyn_gather(ss, p)
            mean = s * inv_d
            var = ss * inv_d - mean * mean
            rstd = _rsqrt_newton(var + EPS)
            onehot = lane == j
            return (jnp.where(onehot, mean, accm),
                    jnp.where(onehot, rstd, accr))

        def compute(b):
            gb, ob = gbuf[b], obuf[b]

            def grp_stats(g, _):
                r0 = g * L
                accm = jnp.zeros((L,), jnp.float32)
                accr = jnp.zeros((L,), jnp.float32)
                for j in range(L):
                    accm, accr = row_stats(gb, r0 + j, j, accm, accr)
                stats_m[pl.ds(r0, L)] = accm
                stats_r[pl.ds(r0, L)] = accr
                return 0

            lax.fori_loop(0, CH // L, grp_stats, 0)

            def grp_norm(g, _):
                r0 = g * L
                mv = stats_m[pl.ds(r0, L)]
                rv = stats_r[pl.ds(r0, L)]
                for j in range(L):
                    jv = jnp.full((L,), j, jnp.int32)
                    mj = _dyn_gather(mv, jv)
                    rj = _dyn_gather(rv, jv)
                    r = r0 + j
                    for k in range(nk):
                        ob[r, pl.ds(k * L, L)] = (
                            (gb[r, pl.ds(k * L, L)] - mj) * rj * gv[k]
                            + bv[k])
                return 0

            lax.fori_loop(0, CH // L, grp_norm, 0)

        issue_gather(0, 0)
        issue_gather(1, 1)

        def pair_body(p, _):
            for b in (0, 1):
                c = p * 2 + b
                wait_gather(b)
                compute(b)
                pl.when(c + 2 < chunks)(lambda: issue_gather(c + 2, b))
                pl.when(c >= 2)(lambda: wait_out(b))
                issue_out(c, b)
            return 0

        lax.fori_loop(0, chunks // 2, pair_body, 0)
        wait_out(0)
        wait_out(1)

    return kern


def kernel(src, table, gamma, beta):
    b, s = src.shape
    v, d = table.shape
    idx = src.reshape(-1).astype(jnp.int32)
    n = idx.shape[0]
    out = _make_sc_kernel(n, d)(table, idx, gamma, beta)
    return out.reshape(b, s, d)


# speedup vs baseline: 1.6182x; 1.5360x over previous
import functools

import jax
import jax.numpy as jnp
from jax import lax
from jax.experimental import pallas as pl
from jax.experimental.pallas import tpu as pltpu
from jax.experimental.pallas import tpu_sc as plsc

EPS = 1e-6
L = 16
NW = 32
BCH = 4

_GATHER_DNUMS = lax.GatherDimensionNumbers(
    offset_dims=(), collapsed_slice_dims=(0,), start_index_map=(0,))


def _dyn_gather(x, idx):
    return lax.gather(
        x, idx[:, None], _GATHER_DNUMS, slice_sizes=(1,),
        mode=lax.GatherScatterMode.PROMISE_IN_BOUNDS)


def _rsqrt_newton(xv):
    i = lax.bitcast_convert_type(xv, jnp.int32)
    i = jnp.int32(0x5F375A86) - lax.shift_right_arithmetic(i, jnp.int32(1))
    y = lax.bitcast_convert_type(i, jnp.float32)
    half = xv * 0.5
    for _ in range(2):
        y = y * (1.5 - half * y * y)
    return y


def _make_sc_kernel(nb, seq, dim):
    b_per_w = nb // NW
    chunks = b_per_w // BCH
    rows = BCH * seq
    n_grp = rows // L
    tail = rows - n_grp * L
    stats_len = (n_grp + (1 if tail else 0)) * L
    nk = dim // L
    mesh = plsc.VectorSubcoreMesh(core_axis_name="c", subcore_axis_name="s")

    @functools.partial(
        pl.kernel,
        out_type=jax.ShapeDtypeStruct((nb, seq, dim), jnp.float32),
        mesh=mesh,
        scratch_types=[
            pltpu.VMEM((b_per_w, seq), jnp.int32),
            pltpu.VMEM((rows, dim), jnp.float32),
            pltpu.VMEM((rows, dim), jnp.float32),
            pltpu.VMEM((rows, dim), jnp.float32),
            pltpu.VMEM((rows, dim), jnp.float32),
            pltpu.VMEM((dim,), jnp.float32),
            pltpu.VMEM((dim,), jnp.float32),
            pltpu.VMEM((stats_len,), jnp.float32),
            pltpu.VMEM((stats_len,), jnp.float32),
            pltpu.SemaphoreType.DMA,
            pltpu.SemaphoreType.DMA,
            pltpu.SemaphoreType.DMA,
            pltpu.SemaphoreType.DMA,
        ],
    )
    def kern(table_hbm, idx_hbm, gamma_hbm, beta_hbm, out_hbm,
             idx_v, gb0, gb1, ob0, ob1, gamma_v, beta_v,
             stats_m, stats_r, gs0, gs1, os0, os1):
        wid = lax.axis_index("s") * 2 + lax.axis_index("c")
        b0w = wid * b_per_w
        pltpu.sync_copy(idx_hbm.at[pl.ds(b0w, b_per_w)], idx_v)
        pltpu.sync_copy(gamma_hbm, gamma_v)
        pltpu.sync_copy(beta_hbm, beta_v)
        gv = [gamma_v[pl.ds(k * L, L)] for k in range(nk)]
        bv = [beta_v[pl.ds(k * L, L)] for k in range(nk)]
        inv_d = jnp.float32(1.0 / dim)
        lane = lax.iota(jnp.int32, L)
        perms = [lane ^ sh for sh in (8, 4, 2, 1)]

        gbuf = (gb0, gb1)
        obuf = (ob0, ob1)
        gsem = (gs0, gs1)
        osem = (os0, os1)

        def issue_gather(c, b):
            for i in range(BCH):
                pltpu.async_copy(
                    table_hbm.at[idx_v.at[c * BCH + i]],
                    gbuf[b].at[pl.ds(i * seq, seq)], gsem[b])

        def wait_gather(b):
            pltpu.make_async_copy(table_hbm.at[pl.ds(0, rows)], gbuf[b],
                                  gsem[b]).wait()

        def issue_out(c, b):
            for i in range(BCH):
                pltpu.async_copy(obuf[b].at[pl.ds(i * seq, seq)],
                                 out_hbm.at[b0w + c * BCH + i], osem[b])

        def wait_out(b):
            for i in range(BCH):
                pltpu.make_async_copy(obuf[b].at[pl.ds(i * seq, seq)],
                                      out_hbm.at[0], osem[b]).wait()

        def row_stats(gb, r, j, accm, accr):
            xs = [gb[r, pl.ds(k * L, L)] for k in range(nk)]
            s01, s23 = xs[0] + xs[1], xs[2] + xs[3]
            s45, s67 = xs[4] + xs[5], xs[6] + xs[7]
            s = (s01 + s23) + (s45 + s67)
            sq = [x * x for x in xs]
            q01, q23 = sq[0] + sq[1], sq[2] + sq[3]
            q45, q67 = sq[4] + sq[5], sq[6] + sq[7]
            ss = (q01 + q23) + (q45 + q67)
            for p in perms:
                s = s + _dyn_gather(s, p)
                ss = ss + _dyn_gather(ss, p)
            mean = s * inv_d
            var = ss * inv_d - mean * mean
            rstd = _rsqrt_newton(var + EPS)
            onehot = lane == j
            return (jnp.where(onehot, mean, accm),
                    jnp.where(onehot, rstd, accr))

        def norm_row(gb, ob, mv, rv, r, j):
            jv = jnp.full((L,), j, jnp.int32)
            mj = _dyn_gather(mv, jv)
            rj = _dyn_gather(rv, jv)
            for k in range(nk):
                ob[r, pl.ds(k * L, L)] = (
                    (gb[r, pl.ds(k * L, L)] - mj) * rj * gv[k] + bv[k])

        def compute(b):
            gb, ob = gbuf[b], obuf[b]

            def grp_stats(g, _):
                r0 = g * L
                accm = jnp.zeros((L,), jnp.float32)
                accr = jnp.zeros((L,), jnp.float32)
                for j in range(L):
                    accm, accr = row_stats(gb, r0 + j, j, accm, accr)
                stats_m[pl.ds(r0, L)] = accm
                stats_r[pl.ds(r0, L)] = accr
                return 0

            lax.fori_loop(0, n_grp, grp_stats, 0)
            if tail:
                r0 = n_grp * L
                accm = jnp.zeros((L,), jnp.float32)
                accr = jnp.zeros((L,), jnp.float32)
                for j in range(tail):
                    accm, accr = row_stats(gb, r0 + j, j, accm, accr)
                stats_m[pl.ds(r0, L)] = accm
                stats_r[pl.ds(r0, L)] = accr

            def grp_norm(g, _):
                r0 = g * L
                mv = stats_m[pl.ds(r0, L)]
                rv = stats_r[pl.ds(r0, L)]
                for j in range(L):
                    norm_row(gb, ob, mv, rv, r0 + j, j)
                return 0

            lax.fori_loop(0, n_grp, grp_norm, 0)
            if tail:
                r0 = n_grp * L
                mv = stats_m[pl.ds(r0, L)]
                rv = stats_r[pl.ds(r0, L)]
                for j in range(tail):
                    norm_row(gb, ob, mv, rv, r0 + j, j)

        issue_gather(0, 0)
        issue_gather(1, 1)

        def pair_body(p, _):
            for b in (0, 1):
                c = p * 2 + b
                wait_gather(b)
                compute(b)
                pl.when(c + 2 < chunks)(lambda: issue_gather(c + 2, b))
                pl.when(c >= 2)(lambda: wait_out(b))
                issue_out(c, b)
            return 0

        lax.fori_loop(0, chunks // 2, pair_body, 0)
        wait_out(0)
        wait_out(1)

    return kern


def kernel(src, table, gamma, beta):
    b, s = src.shape
    v, d = table.shape
    return _make_sc_kernel(b, s, d)(table, src.astype(jnp.int32),
                                    gamma, beta)


# speedup vs baseline: 1.6197x; 1.0009x over previous
import functools

import jax
import jax.numpy as jnp
from jax import lax
from jax.experimental import pallas as pl
from jax.experimental.pallas import tpu as pltpu
from jax.experimental.pallas import tpu_sc as plsc

EPS = 1e-6
L = 16
NW = 32
BCH = 4

_GATHER_DNUMS = lax.GatherDimensionNumbers(
    offset_dims=(), collapsed_slice_dims=(0,), start_index_map=(0,))


def _dyn_gather(x, idx):
    return lax.gather(
        x, idx[:, None], _GATHER_DNUMS, slice_sizes=(1,),
        mode=lax.GatherScatterMode.PROMISE_IN_BOUNDS)


def _rsqrt_newton(xv):
    i = lax.bitcast_convert_type(xv, jnp.int32)
    i = jnp.int32(0x5F375A86) - lax.shift_right_arithmetic(i, jnp.int32(1))
    y = lax.bitcast_convert_type(i, jnp.float32)
    half = xv * 0.5
    y = y * (1.5 - half * y * y)
    y = y * (1.5 - half * y * y)
    return y


def _make_sc_kernel(nb, seq, dim):
    b_per_w = nb // NW
    chunks = b_per_w // BCH
    rows = BCH * seq
    n_grp = rows // L
    tail = rows - n_grp * L
    stats_len = (n_grp + (1 if tail else 0)) * L
    nk = dim // L
    mesh = plsc.VectorSubcoreMesh(core_axis_name="c", subcore_axis_name="s")

    @functools.partial(
        pl.kernel,
        out_type=jax.ShapeDtypeStruct((nb, seq, dim), jnp.float32),
        mesh=mesh,
        scratch_types=[
            pltpu.VMEM((b_per_w * seq,), jnp.int32),
            pltpu.VMEM((rows, dim), jnp.float32),
            pltpu.VMEM((rows, dim), jnp.float32),
            pltpu.VMEM((rows, dim), jnp.float32),
            pltpu.VMEM((rows, dim), jnp.float32),
            pltpu.VMEM((dim,), jnp.float32),
            pltpu.VMEM((dim,), jnp.float32),
            pltpu.VMEM((stats_len,), jnp.float32),
            pltpu.VMEM((stats_len,), jnp.float32),
            pltpu.SemaphoreType.DMA,
            pltpu.SemaphoreType.DMA,
            pltpu.SemaphoreType.DMA,
            pltpu.SemaphoreType.DMA,
        ],
    )
    def kern(table_hbm, idx_hbm, gamma_hbm, beta_hbm, out_hbm,
             idx_v, gb0, gb1, ob0, ob1, gamma_v, beta_v,
             stats_m, stats_r, gs0, gs1, os0, os1):
        wid = lax.axis_index("s") * 2 + lax.axis_index("c")
        b0w = wid * b_per_w
        pltpu.sync_copy(idx_hbm.at[pl.ds(b0w * seq, b_per_w * seq)], idx_v)
        pltpu.sync_copy(gamma_hbm, gamma_v)
        pltpu.sync_copy(beta_hbm, beta_v)
        gv = [gamma_v[pl.ds(k * L, L)] for k in range(nk)]
        bv = [beta_v[pl.ds(k * L, L)] for k in range(nk)]
        inv_d = jnp.float32(1.0 / dim)
        lane = lax.iota(jnp.int32, L)
        perms = [lane ^ sh for sh in (8, 4, 2, 1)]

        gbuf = (gb0, gb1)
        obuf = (ob0, ob1)
        gsem = (gs0, gs1)
        osem = (os0, os1)

        split = min(128, (rows // 2 + 7) & ~7)

        def issue_gather(c, b):
            base = c * rows
            pltpu.async_copy(table_hbm.at[idx_v.at[pl.ds(base, split)]],
                             gbuf[b].at[pl.ds(0, split)], gsem[b])
            pltpu.async_copy(
                table_hbm.at[idx_v.at[pl.ds(base + split, rows - split)]],
                gbuf[b].at[pl.ds(split, rows - split)], gsem[b])

        def wait_gather(b):
            pltpu.make_async_copy(table_hbm.at[pl.ds(0, rows)], gbuf[b],
                                  gsem[b]).wait()

        def issue_out(c, b):
            for i in range(BCH):
                pltpu.async_copy(obuf[b].at[pl.ds(i * seq, seq)],
                                 out_hbm.at[b0w + c * BCH + i], osem[b])

        def wait_out(b):
            for i in range(BCH):
                pltpu.make_async_copy(obuf[b].at[pl.ds(i * seq, seq)],
                                      out_hbm.at[0], osem[b]).wait()

        def row_stats(gb, r, j, accm, accr):
            xs = [gb[r, pl.ds(k * L, L)] for k in range(nk)]
            s01, s23 = xs[0] + xs[1], xs[2] + xs[3]
            s45, s67 = xs[4] + xs[5], xs[6] + xs[7]
            s = (s01 + s23) + (s45 + s67)
            sq = [x * x for x in xs]
            q01, q23 = sq[0] + sq[1], sq[2] + sq[3]
            q45, q67 = sq[4] + sq[5], sq[6] + sq[7]
            ss = (q01 + q23) + (q45 + q67)
            for p in perms:
                s = s + _dyn_gather(s, p)
                ss = ss + _dyn_gather(ss, p)
            mean = s * inv_d
            var = ss * inv_d - mean * mean
            rstd = _rsqrt_newton(var + EPS)
            onehot = lane == j
            return (jnp.where(onehot, mean, accm),
                    jnp.where(onehot, rstd, accr))

        def norm_row(gb, ob, mv, rv, r, j):
            jv = jnp.full((L,), j, jnp.int32)
            mj = _dyn_gather(mv, jv)
            rj = _dyn_gather(rv, jv)
            for k in range(nk):
                ob[r, pl.ds(k * L, L)] = (
                    (gb[r, pl.ds(k * L, L)] - mj) * rj * gv[k] + bv[k])

        def compute(b):
            gb, ob = gbuf[b], obuf[b]

            def grp_stats(g, _):
                r0 = g * L
                accm = jnp.zeros((L,), jnp.float32)
                accr = jnp.zeros((L,), jnp.float32)
                for j in range(L):
                    accm, accr = row_stats(gb, r0 + j, j, accm, accr)
                stats_m[pl.ds(r0, L)] = accm
                stats_r[pl.ds(r0, L)] = accr
                return 0

            lax.fori_loop(0, n_grp, grp_stats, 0)
            if tail:
                r0 = n_grp * L
                accm = jnp.zeros((L,), jnp.float32)
                accr = jnp.zeros((L,), jnp.float32)
                for j in range(tail):
                    accm, accr = row_stats(gb, r0 + j, j, accm, accr)
                stats_m[pl.ds(r0, L)] = accm
                stats_r[pl.ds(r0, L)] = accr

            def grp_norm(g, _):
                r0 = g * L
                mv = stats_m[pl.ds(r0, L)]
                rv = stats_r[pl.ds(r0, L)]
                for j in range(L):
                    norm_row(gb, ob, mv, rv, r0 + j, j)
                return 0

            lax.fori_loop(0, n_grp, grp_norm, 0)
            if tail:
                r0 = n_grp * L
                mv = stats_m[pl.ds(r0, L)]
                rv = stats_r[pl.ds(r0, L)]
                for j in range(tail):
                    norm_row(gb, ob, mv, rv, r0 + j, j)

        issue_gather(0, 0)
        issue_gather(1, 1)

        def pair_body(p, _):
            for b in (0, 1):
                c = p * 2 + b
                wait_gather(b)
                compute(b)
                pl.when(c + 2 < chunks)(lambda: issue_gather(c + 2, b))
                pl.when(c >= 2)(lambda: wait_out(b))
                issue_out(c, b)
            return 0

        lax.fori_loop(0, chunks // 2, pair_body, 0)
        wait_out(0)
        wait_out(1)

    return kern


def kernel(src, table, gamma, beta):
    b, s = src.shape
    v, d = table.shape
    idx = src.reshape(-1).astype(jnp.int32)
    return _make_sc_kernel(b, s, d)(table, idx, gamma, beta)
